# baseline (speedup 1.0000x reference)
import functools

import jax
import jax.numpy as jnp
from jax import lax
from jax.experimental import pallas as pl
from jax.experimental.pallas import tpu as pltpu
from jax.experimental.pallas import tpu_sc as plsc

B, T, WMAX, DM = 4, 512, 50, 200
DI, DS, DTR, K = 400, 16, 13, 4
POS = T // 2
L = T + 1

TC_AGG = 128
T_TC = 384
T_SC = T - T_TC
N_SC = B * T_SC
NWORK = 32
TPW = N_SC // NWORK
WD = WMAX * DM
NCH = 13

_INTERP = False


def _sigmoid(x):
    return 1.0 / (1.0 + jnp.exp(-x))


def _silu(x):
    return x * _sigmoid(x)


def _softplus(x):
    return jnp.maximum(x, 0.0) + jnp.log(1.0 + jnp.exp(-jnp.abs(x)))


def _agg_body(nw_ref, ids_ref, w50_ref, r_ref, s_ref, ba_ref, out_ref):
    ids = ids_ref[0].astype(jnp.bfloat16)
    scores = jax.lax.dot_general(
        ids, w50_ref[...], (((1,), (0,)), ((), ())),
        preferred_element_type=jnp.float32)
    scores = scores + jnp.broadcast_to(ba_ref[0:1, 0:1], (TC_AGG, WMAX))
    nw = nw_ref[0, 0]
    wm = jax.lax.broadcasted_iota(jnp.int32, (TC_AGG, WMAX), 1) < nw[:, None]
    scores = jnp.where(wm, scores, -1e30)
    m = jnp.max(scores, axis=1, keepdims=True)
    e = jnp.exp(scores - m)
    attn = e / jnp.sum(e, axis=1, keepdims=True)
    attn = jnp.where(wm, attn, 0.0).astype(jnp.bfloat16)
    attn_e = jax.lax.dot_general(
        attn, r_ref[...], (((1,), (0,)), ((), ())),
        preferred_element_type=jnp.float32).astype(jnp.bfloat16)
    out_ref[0] = jax.lax.dot_general(
        ids * attn_e, s_ref[...], (((1,), (0,)), ((), ())),
        preferred_element_type=jnp.float32)


def _sc_agg_body(ids_hbm, nw_hbm, wa_hbm, out_hbm, buf, nwv_s, wab, outg,
                 redA, redB, redC):
    wid = lax.axis_index("s") * 2 + lax.axis_index("c")
    bidx = wid // (T_SC // TPW)
    base_row = bidx * T + T_TC + (wid % (T_SC // TPW)) * TPW
    base_out = wid * TPW

    pltpu.sync_copy(nw_hbm.at[pl.ds(base_row, TPW)], nwv_s)
    pltpu.sync_copy(wa_hbm, wab)
    wav = [wab[pl.ds(16 * c, 16)] for c in range(NCH)]
    lane = lax.iota(jnp.int32, 16)
    lane_f = lane.astype(jnp.float32)
    NEG = jnp.float32(-1e30)
    nwf_vec = nwv_s[...].astype(jnp.float32)
    nws = [nwf_vec[i] for i in range(TPW)]

    redA[pl.ds(16, 16)] = jnp.zeros((16,), jnp.float32)
    redB[pl.ds(0, 16)] = jnp.full((16,), NEG, jnp.float32)
    redC[pl.ds(16, 16)] = jnp.full((16,), NEG, jnp.float32)

    def _fold_sum(v):
        for off in (8, 4, 2, 1):
            redA[pl.ds(0, 16)] = v
            v = v + redA[pl.ds(off, 16)]
        return v

    def _bcast_max(v):
        for off in (1, 2, 4, 8):
            redB[pl.ds(16, 16)] = v
            v = jnp.maximum(v, redB[pl.ds(16 - off, 16)])
        return v

    def _fold_max(v):
        for off in (8, 4, 2, 1):
            redC[pl.ds(0, 16)] = v
            v = jnp.maximum(v, redC[pl.ds(off, 16)])
        return v

    def tweet_body(j, carry):
        row = base_row + j
        pltpu.sync_copy(ids_hbm.at[row], buf)
        nw_f = jnp.float32(0.0)
        for i in range(TPW):
            nw_f = nw_f + jnp.where(j == i, nws[i], 0.0)

        svecs = [jnp.full((16,), NEG, jnp.float32) for _ in range(4)]
        for w in range(WMAX):
            acc = buf[w, pl.ds(0, 16)] * wav[0]
            for c in range(1, NCH - 1):
                acc = acc + buf[w, pl.ds(16 * c, 16)] * wav[c]
            acc = acc + buf[w, pl.ds(DM - 16, 16)] * wav[NCH - 1]
            s_w = _fold_sum(acc)[0]
            g, sl = divmod(w, 16)
            svecs[g] = jnp.where(lane == sl, s_w, svecs[g])

        masked = []
        for g in range(4):
            wm = (lane_f + 16.0 * g) < nw_f
            masked.append(jnp.where(wm, svecs[g], NEG))
        mv = jnp.maximum(jnp.maximum(masked[0], masked[1]),
                         jnp.maximum(masked[2], masked[3]))
        mv = _bcast_max(_fold_max(mv))
        evs = []
        for g in range(4):
            wm = (lane_f + 16.0 * g) < nw_f
            evs.append(jnp.where(wm, jnp.exp(masked[g] - mv), 0.0))
        es = evs[0] + evs[1] + evs[2] + evs[3]
        es = _bcast_max(_fold_sum(es))
        invv = jnp.ones((16,), jnp.float32) / es
        avecs = [e * invv for e in evs]

        accs = [jnp.zeros((16,), jnp.float32) for _ in range(NCH)]
        for w in range(WMAX):
            a_w = avecs[w // 16][w % 16]
            for c in range(NCH - 1):
                accs[c] = accs[c] + a_w * buf[w, pl.ds(16 * c, 16)]
            accs[NCH - 1] = accs[NCH - 1] + a_w * buf[w, pl.ds(DM - 16, 16)]
        for c in range(NCH - 1):
            outg[pl.ds(DM * j + 16 * c, 16)] = accs[c]
        outg[pl.ds(DM * j + DM - 16, 16)] = accs[NCH - 1]
        return carry

    lax.fori_loop(0, TPW, tweet_body, 0)
    pltpu.sync_copy(outg, out_hbm.at[pl.ds(base_out * DM, TPW * DM)])


def _sc_agg(ids_flat, nw_flat, wa_pad):
    mesh = plsc.VectorSubcoreMesh(core_axis_name="c", subcore_axis_name="s")
    kern = functools.partial(
        pl.kernel,
        mesh=mesh,
        out_type=jax.ShapeDtypeStruct((N_SC * DM,), jnp.float32),
        scratch_types=[
            pltpu.VMEM((WMAX, DM), jnp.float32),
            pltpu.VMEM((TPW,), jnp.int32),
            pltpu.VMEM((16 * NCH,), jnp.float32),
            pltpu.VMEM((TPW * DM,), jnp.float32),
            pltpu.VMEM((32,), jnp.float32),
            pltpu.VMEM((32,), jnp.float32),
            pltpu.VMEM((32,), jnp.float32),
        ],
    )(_sc_agg_body)
    return kern(ids_flat, nw_flat, wa_pad)


def _mamba_body(emb_ref, embsc_ref, nt_ref, cls_ref, normw_ref, inW_ref,
                convw_ref, convb_ref, xW_ref, dtw_ref, dtb_ref, Alog_ref,
                Dssm_ref, outW_ref, headw_ref, headb_ref, out_ref):
    convw = convw_ref[...]
    convb = convb_ref[0][None, :]
    dtb = dtb_ref[0][None, :]
    normw = normw_ref[0][None, :]
    cls = cls_ref[...]
    AT = -jnp.exp(jnp.transpose(Alog_ref[...]))

    W = POS + 1
    ri = jax.lax.broadcasted_iota(jnp.int32, (W, W), 0)
    ci = jax.lax.broadcasted_iota(jnp.int32, (W, W), 1)
    TF = jnp.where(ci > ri, 1.0, 0.0)
    TB = jnp.where(ci < ri, 1.0, 0.0)

    grows = []
    z_pos = []
    for b in range(B):
        nt_b = jnp.broadcast_to(nt_ref[0:1, b:b + 1], (T, DM))
        emb_b = jnp.concatenate([emb_ref[b], embsc_ref[b]], axis=0)
        tmask = jax.lax.broadcasted_iota(jnp.int32, (T, DM), 0) < nt_b
        x0 = jnp.where(tmask, emb_b, 0.0)
        x_b = jnp.concatenate([x0[:POS], cls, x0[POS:]], axis=0)
        h_b = x_b * jax.lax.rsqrt(
            jnp.mean(x_b * x_b, axis=-1, keepdims=True) + 1e-5) * normw
        xz_b = h_b @ inW_ref[...]
        xs0 = xz_b[:, :DI]
        z_pos.append(xz_b[POS:POS + 1, DI:])

        zpad = jnp.zeros((K - 1, DI), jnp.float32)
        xpF = jnp.concatenate([zpad, xs0], axis=0)
        xpB = jnp.concatenate([xs0, zpad], axis=0)
        accF = convb
        accB = convb
        for k in range(K):
            wk = convw[:, k][None, :]
            accF = accF + xpF[k:k + L] * wk
            accB = accB + xpB[K - 1 - k:K - 1 - k + L] * wk
        xsF_b = _silu(accF)
        xsB_b = _silu(accB)

        dssm = Dssm_ref[0][None, :]
        ysum = jnp.zeros((1, DI), jnp.float32)
        for xs_b, TRI, w0, pos_row in (
                (xsF_b, TF, 0, W - 1),
                (xsB_b, TB, POS, 0)):
            dbc_b = xs_b @ xW_ref[...]
            dt_b = _softplus(dbc_b[:, :DTR] @ dtw_ref[...] + dtb)
            dtw_w = dt_b[w0:w0 + W]
            duw = dtw_w * xs_b[w0:w0 + W]
            bmw = dbc_b[w0:w0 + W, DTR:DTR + DS]
            cs = TRI @ dtw_w
            y = jnp.zeros((1, DI), jnp.float32)
            for s in range(DS):
                cms = dbc_b[w0 + pos_row:w0 + pos_row + 1,
                            DTR + DS + s:DTR + DS + s + 1]
                wcol = bmw[:, s:s + 1] * cms
                contrib = jnp.exp(cs * AT[s:s + 1, :]) * duw * wcol
                y = y + jnp.sum(contrib, axis=0, keepdims=True)
            ysum = ysum + y + xs_b[POS:POS + 1] * dssm
        grows.append(ysum * _silu(z_pos[b]))
    G = jnp.concatenate(grows, axis=0)
    outp = G @ outW_ref[...]
    xfin = jnp.broadcast_to(cls, (B, DM)) + outp
    logits = xfin @ headw_ref[...] + jnp.broadcast_to(headb_ref[0:1, 0:1], (B, 1))
    out_ref[...] = _sigmoid(logits)


def kernel(input_ids, cls_token, W_attn, b_attn, norm_w, in_proj_W, conv_w,
           conv_b, x_proj_W, dt_w, dt_b, A_log, D_ssm, out_proj_W, head_w,
           head_b, n_tweets, n_words):
    nw3 = jnp.reshape(n_words, (B, 1, T)).astype(jnp.int32)
    ba2 = jnp.reshape(b_attn, (1, 1))
    nt2 = jnp.reshape(n_tweets, (1, B)).astype(jnp.int32)
    cb2 = jnp.reshape(conv_b, (1, DI))
    dtb2 = jnp.reshape(dt_b, (1, DI))
    dssm2 = jnp.reshape(D_ssm, (1, DI))
    normw2 = jnp.reshape(norm_w, (1, DM))
    hb2 = jnp.reshape(head_b, (1, 1))

    ids2 = jnp.reshape(input_ids, (B, T, WMAX * DM))
    eyew = jnp.eye(WMAX, dtype=jnp.float32)
    w50 = jnp.reshape(eyew[:, None, :] * W_attn[:, 0][None, :, None],
                      (WMAX * DM, WMAX)).astype(jnp.bfloat16)
    rexp = jnp.reshape(jnp.broadcast_to(eyew[:, :, None], (WMAX, WMAX, DM)),
                       (WMAX, WMAX * DM)).astype(jnp.bfloat16)
    ssum = jnp.reshape(jnp.broadcast_to(jnp.eye(DM, dtype=jnp.float32)[None],
                                        (WMAX, DM, DM)),
                       (WMAX * DM, DM)).astype(jnp.bfloat16)
    emb = pl.pallas_call(
        _agg_body,
        grid=(B, T_TC // TC_AGG),
        in_specs=[
            pl.BlockSpec((1, 1, TC_AGG), lambda b, t: (b, 0, t)),
            pl.BlockSpec((1, TC_AGG, WMAX * DM), lambda b, t: (b, t, 0)),
            pl.BlockSpec((WMAX * DM, WMAX), lambda b, t: (0, 0)),
            pl.BlockSpec((WMAX, WMAX * DM), lambda b, t: (0, 0)),
            pl.BlockSpec((WMAX * DM, DM), lambda b, t: (0, 0)),
            pl.BlockSpec((1, 1), lambda b, t: (0, 0)),
        ],
        out_specs=pl.BlockSpec((1, TC_AGG, DM), lambda b, t: (b, t, 0)),
        out_shape=jax.ShapeDtypeStruct((B, T_TC, DM), jnp.float32),
        interpret=_INTERP,
    )(nw3, ids2, w50, rexp, ssum, ba2)

    ids3 = jnp.reshape(input_ids, (B * T, WMAX, DM))
    nw_flat = jnp.reshape(n_words, (B * T,)).astype(jnp.int32)
    wa_sc = jnp.concatenate(
        [W_attn[:DM - 16, 0], jnp.zeros((8,), jnp.float32),
         W_attn[DM - 16:DM - 8, 0], W_attn[DM - 8:, 0]])
    emb_sc = jnp.reshape(_sc_agg(ids3, nw_flat, wa_sc), (B, T_SC, DM))

    out = pl.pallas_call(
        _mamba_body,
        out_shape=jax.ShapeDtypeStruct((B, 1), jnp.float32),
        interpret=_INTERP,
    )(emb, emb_sc, nt2, jnp.reshape(cls_token, (1, DM)), normw2, in_proj_W,
      conv_w, cb2, x_proj_W, dt_w, dtb2, A_log, dssm2, out_proj_W, head_w,
      hb2)

    return jnp.reshape(out, (B,))


# speedup vs baseline: 1.4102x; 1.4102x over previous
import functools

import jax
import jax.numpy as jnp
from jax import lax
from jax.experimental import pallas as pl
from jax.experimental.pallas import tpu as pltpu
from jax.experimental.pallas import tpu_sc as plsc

B, T, WMAX, DM = 4, 512, 50, 200
DI, DS, DTR, K = 400, 16, 13, 4
POS = T // 2
L = T + 1

TC_AGG = 128
T_TC = 512
T_SC = T - T_TC
N_SC = B * T_SC
NWORK = 32
TPW = N_SC // NWORK
WD = WMAX * DM
NCH = 13

_INTERP = False


def _sigmoid(x):
    return 1.0 / (1.0 + jnp.exp(-x))


def _silu(x):
    return x * _sigmoid(x)


def _softplus(x):
    return jnp.maximum(x, 0.0) + jnp.log(1.0 + jnp.exp(-jnp.abs(x)))


def _agg_body(nw_ref, ids_ref, w50_ref, r_ref, s_ref, ba_ref, out_ref):
    ids = ids_ref[0].astype(jnp.bfloat16)
    scores = jax.lax.dot_general(
        ids, w50_ref[...], (((1,), (0,)), ((), ())),
        preferred_element_type=jnp.float32)
    scores = scores + jnp.broadcast_to(ba_ref[0:1, 0:1], (TC_AGG, WMAX))
    nw = nw_ref[0, 0]
    wm = jax.lax.broadcasted_iota(jnp.int32, (TC_AGG, WMAX), 1) < nw[:, None]
    scores = jnp.where(wm, scores, -1e30)
    m = jnp.max(scores, axis=1, keepdims=True)
    e = jnp.exp(scores - m)
    attn = e / jnp.sum(e, axis=1, keepdims=True)
    attn = jnp.where(wm, attn, 0.0).astype(jnp.bfloat16)
    attn_e = jax.lax.dot_general(
        attn, r_ref[...], (((1,), (0,)), ((), ())),
        preferred_element_type=jnp.float32).astype(jnp.bfloat16)
    out_ref[0] = jax.lax.dot_general(
        ids * attn_e, s_ref[...], (((1,), (0,)), ((), ())),
        preferred_element_type=jnp.float32)


def _sc_agg_body(ids_hbm, nw_hbm, wa_hbm, out_hbm, buf, nwv_s, wab, outg,
                 sacc, evb, red):
    wid = lax.axis_index("s") * 2 + lax.axis_index("c")
    bidx = wid // (T_SC // TPW)
    base_row = bidx * T + T_TC + (wid % (T_SC // TPW)) * TPW
    base_out = wid * TPW

    pltpu.sync_copy(nw_hbm.at[pl.ds(base_row, TPW)], nwv_s)
    pltpu.sync_copy(wa_hbm, wab)
    wav = [wab[pl.ds(16 * c, 16)] for c in range(NCH)]
    lane = lax.iota(jnp.int32, 16)
    lane_f = lane.astype(jnp.float32)
    NEG = jnp.float32(-1e30)
    nwf_vec = nwv_s[...].astype(jnp.float32)
    nws = [nwf_vec[i] for i in range(TPW)]

    def tweet_body(j, carry):
        row = base_row + j
        pltpu.sync_copy(ids_hbm.at[row], buf)
        nw_f = jnp.float32(0.0)
        for i in range(TPW):
            nw_f = nw_f + jnp.where(j == i, nws[i], 0.0)

        def score_w(w, c_):
            acc = buf[w, pl.ds(0, 16)] * wav[0]
            for c in range(1, NCH - 1):
                acc = acc + buf[w, pl.ds(16 * c, 16)] * wav[c]
            acc = acc + buf[w, pl.ds(DM - 16, 16)] * wav[NCH - 1]
            sacc[pl.ds(16 * w, 16)] = acc
            return c_
        lax.fori_loop(0, WMAX, score_w, 0)

        svecs = []
        for g in range(4):
            base_idx = lane * 16 + 256 * g
            t = plsc.load_gather(sacc, [base_idx])
            for c in range(1, 16):
                t = t + plsc.load_gather(sacc, [base_idx + c])
            svecs.append(t)

        masked = []
        for g in range(4):
            wm = (lane_f + 16.0 * g) < nw_f
            masked.append(jnp.where(wm, svecs[g], NEG))
        mv = jnp.maximum(jnp.maximum(masked[0], masked[1]),
                         jnp.maximum(masked[2], masked[3]))
        for sh in (8, 4, 2, 1):
            red[...] = mv
            mv = jnp.maximum(mv, plsc.load_gather(red, [lane ^ sh]))
        evs = []
        for g in range(4):
            wm = (lane_f + 16.0 * g) < nw_f
            evs.append(jnp.where(wm, jnp.exp(masked[g] - mv), 0.0))
        es = evs[0] + evs[1] + evs[2] + evs[3]
        for sh in (8, 4, 2, 1):
            red[...] = es
            es = es + plsc.load_gather(red, [lane ^ sh])
        invv = jnp.ones((16,), jnp.float32) / es
        for g in range(4):
            evb[pl.ds(16 * g, 16)] = evs[g] * invv

        def wsum_w(w, accs):
            a_vec = plsc.load_gather(evb, [jnp.broadcast_to(w, (16,))])
            new = [accs[c] + a_vec * buf[w, pl.ds(16 * c, 16)]
                   for c in range(NCH - 1)]
            new.append(accs[NCH - 1] + a_vec * buf[w, pl.ds(DM - 16, 16)])
            return tuple(new)
        accs = lax.fori_loop(0, WMAX, wsum_w,
                             tuple(jnp.zeros((16,), jnp.float32)
                                   for _ in range(NCH)))
        for c in range(NCH - 1):
            outg[pl.ds(DM * j + 16 * c, 16)] = accs[c]
        outg[pl.ds(DM * j + DM - 16, 16)] = accs[NCH - 1]
        return carry

    lax.fori_loop(0, TPW, tweet_body, 0)
    pltpu.sync_copy(outg, out_hbm.at[pl.ds(base_out * DM, TPW * DM)])


def _sc_agg(ids_flat, nw_flat, wa_pad):
    mesh = plsc.VectorSubcoreMesh(core_axis_name="c", subcore_axis_name="s")
    kern = functools.partial(
        pl.kernel,
        mesh=mesh,
        out_type=jax.ShapeDtypeStruct((N_SC * DM,), jnp.float32),
        scratch_types=[
            pltpu.VMEM((WMAX, DM), jnp.float32),
            pltpu.VMEM((TPW,), jnp.int32),
            pltpu.VMEM((16 * NCH,), jnp.float32),
            pltpu.VMEM((TPW * DM,), jnp.float32),
            pltpu.VMEM((1024,), jnp.float32),
            pltpu.VMEM((64,), jnp.float32),
            pltpu.VMEM((16,), jnp.float32),
        ],
    )(_sc_agg_body)
    return kern(ids_flat, nw_flat, wa_pad)


def _mamba_body(emb_ref, nt_ref, cls_ref, normw_ref, inW_ref,
                convw_ref, convb_ref, xW_ref, dtw_ref, dtb_ref, Alog_ref,
                Dssm_ref, outW_ref, headw_ref, headb_ref, out_ref):
    convw = convw_ref[...]
    convb = convb_ref[0][None, :]
    dtb = dtb_ref[0][None, :]
    normw = normw_ref[0][None, :]
    cls = cls_ref[...]
    AT = -jnp.exp(jnp.transpose(Alog_ref[...]))

    W = POS + 1
    ri = jax.lax.broadcasted_iota(jnp.int32, (W, W), 0)
    ci = jax.lax.broadcasted_iota(jnp.int32, (W, W), 1)
    TF = jnp.where(ci > ri, 1.0, 0.0)
    TB = jnp.where(ci < ri, 1.0, 0.0)

    grows = []
    z_pos = []
    for b in range(B):
        nt_b = jnp.broadcast_to(nt_ref[0:1, b:b + 1], (T, DM))
        emb_b = emb_ref[b]
        tmask = jax.lax.broadcasted_iota(jnp.int32, (T, DM), 0) < nt_b
        x0 = jnp.where(tmask, emb_b, 0.0)
        x_b = jnp.concatenate([x0[:POS], cls, x0[POS:]], axis=0)
        h_b = x_b * jax.lax.rsqrt(
            jnp.mean(x_b * x_b, axis=-1, keepdims=True) + 1e-5) * normw
        xz_b = h_b @ inW_ref[...]
        xs0 = xz_b[:, :DI]
        z_pos.append(xz_b[POS:POS + 1, DI:])

        zpad = jnp.zeros((K - 1, DI), jnp.float32)
        xpF = jnp.concatenate([zpad, xs0], axis=0)
        xpB = jnp.concatenate([xs0, zpad], axis=0)
        accF = convb
        accB = convb
        for k in range(K):
            wk = convw[:, k][None, :]
            accF = accF + xpF[k:k + L] * wk
            accB = accB + xpB[K - 1 - k:K - 1 - k + L] * wk
        xsF_b = _silu(accF)
        xsB_b = _silu(accB)

        dssm = Dssm_ref[0][None, :]
        ysum = jnp.zeros((1, DI), jnp.float32)
        for xs_b, TRI, w0, pos_row in (
                (xsF_b, TF, 0, W - 1),
                (xsB_b, TB, POS, 0)):
            dbc_b = xs_b @ xW_ref[...]
            dt_b = _softplus(dbc_b[:, :DTR] @ dtw_ref[...] + dtb)
            dtw_w = dt_b[w0:w0 + W]
            duw = dtw_w * xs_b[w0:w0 + W]
            bmw = dbc_b[w0:w0 + W, DTR:DTR + DS]
            cs = TRI @ dtw_w
            y = jnp.zeros((1, DI), jnp.float32)
            for s in range(DS):
                cms = dbc_b[w0 + pos_row:w0 + pos_row + 1,
                            DTR + DS + s:DTR + DS + s + 1]
                wcol = bmw[:, s:s + 1] * cms
                contrib = jnp.exp(cs * AT[s:s + 1, :]) * duw * wcol
                y = y + jnp.sum(contrib, axis=0, keepdims=True)
            ysum = ysum + y + xs_b[POS:POS + 1] * dssm
        grows.append(ysum * _silu(z_pos[b]))
    G = jnp.concatenate(grows, axis=0)
    outp = G @ outW_ref[...]
    xfin = jnp.broadcast_to(cls, (B, DM)) + outp
    logits = xfin @ headw_ref[...] + jnp.broadcast_to(headb_ref[0:1, 0:1], (B, 1))
    out_ref[...] = _sigmoid(logits)


def kernel(input_ids, cls_token, W_attn, b_attn, norm_w, in_proj_W, conv_w,
           conv_b, x_proj_W, dt_w, dt_b, A_log, D_ssm, out_proj_W, head_w,
           head_b, n_tweets, n_words):
    nw3 = jnp.reshape(n_words, (B, 1, T)).astype(jnp.int32)
    ba2 = jnp.reshape(b_attn, (1, 1))
    nt2 = jnp.reshape(n_tweets, (1, B)).astype(jnp.int32)
    cb2 = jnp.reshape(conv_b, (1, DI))
    dtb2 = jnp.reshape(dt_b, (1, DI))
    dssm2 = jnp.reshape(D_ssm, (1, DI))
    normw2 = jnp.reshape(norm_w, (1, DM))
    hb2 = jnp.reshape(head_b, (1, 1))

    ids2 = jnp.reshape(input_ids, (B, T, WMAX * DM))
    eyew = jnp.eye(WMAX, dtype=jnp.float32)
    w50 = jnp.reshape(eyew[:, None, :] * W_attn[:, 0][None, :, None],
                      (WMAX * DM, WMAX)).astype(jnp.bfloat16)
    rexp = jnp.reshape(jnp.broadcast_to(eyew[:, :, None], (WMAX, WMAX, DM)),
                       (WMAX, WMAX * DM)).astype(jnp.bfloat16)
    ssum = jnp.reshape(jnp.broadcast_to(jnp.eye(DM, dtype=jnp.float32)[None],
                                        (WMAX, DM, DM)),
                       (WMAX * DM, DM)).astype(jnp.bfloat16)
    emb = pl.pallas_call(
        _agg_body,
        grid=(B, T_TC // TC_AGG),
        in_specs=[
            pl.BlockSpec((1, 1, TC_AGG), lambda b, t: (b, 0, t)),
            pl.BlockSpec((1, TC_AGG, WMAX * DM), lambda b, t: (b, t, 0)),
            pl.BlockSpec((WMAX * DM, WMAX), lambda b, t: (0, 0)),
            pl.BlockSpec((WMAX, WMAX * DM), lambda b, t: (0, 0)),
            pl.BlockSpec((WMAX * DM, DM), lambda b, t: (0, 0)),
            pl.BlockSpec((1, 1), lambda b, t: (0, 0)),
        ],
        out_specs=pl.BlockSpec((1, TC_AGG, DM), lambda b, t: (b, t, 0)),
        out_shape=jax.ShapeDtypeStruct((B, T_TC, DM), jnp.float32),
        interpret=_INTERP,
    )(nw3, ids2, w50, rexp, ssum, ba2)


    out = pl.pallas_call(
        _mamba_body,
        out_shape=jax.ShapeDtypeStruct((B, 1), jnp.float32),
        interpret=_INTERP,
    )(emb, nt2, jnp.reshape(cls_token, (1, DM)), normw2, in_proj_W,
      conv_w, cb2, x_proj_W, dt_w, dtb2, A_log, dssm2, out_proj_W, head_w,
      hb2)

    return jnp.reshape(out, (B,))
